# baseline (speedup 1.0000x reference)
import functools

import jax
import jax.numpy as jnp
from jax import lax
from jax.experimental import pallas as pl
from jax.experimental.pallas import tpu as pltpu
from jax.experimental.pallas import tpu_sc as plsc

N_OUT_FRAMES = 8 * 16
N_TILES = 32
FRAMES_PER_TILE = N_OUT_FRAMES // N_TILES
LANES = 16


def _sc_gather(x_m, gidx):
  mesh = plsc.VectorSubcoreMesh(core_axis_name="c", subcore_axis_name="s")
  frame_shape = x_m.shape[1:]

  @functools.partial(
      pl.kernel,
      mesh=mesh,
      out_type=jax.ShapeDtypeStruct((N_OUT_FRAMES,) + frame_shape,
                                    jnp.float32),
      scratch_types=[
          pltpu.VMEM((FRAMES_PER_TILE, LANES), jnp.int32),
          pltpu.SemaphoreType.DMA,
      ],
  )
  def k(x_hbm, gidx_hbm, out_hbm, idx_v, sem):
    wid = lax.axis_index("s") * 2 + lax.axis_index("c")
    j0 = wid * FRAMES_PER_TILE
    pltpu.sync_copy(gidx_hbm.at[pl.ds(j0, FRAMES_PER_TILE)], idx_v)
    copies = []
    for i in range(FRAMES_PER_TILE):
      t = idx_v[i, :][0]
      copies.append(
          pltpu.async_copy(x_hbm.at[t], out_hbm.at[j0 + i], sem))
    for c in copies:
      c.wait()

  return k(x_m, gidx)


def kernel(x):
  B, T = x.shape[0], x.shape[1]
  fnum = 16
  start, end = 0, T - 1
  fid_base = jnp.linspace(start, end, fnum).astype(jnp.int32)
  step = int((end - start) / fnum)
  if step != 0:
    key = jax.random.key(42)
    y = jax.random.randint(key, (B, fnum), 0, step, dtype=jnp.int32)
    y = y.at[:, fnum - 1].set(0)
  else:
    y = jnp.zeros((B, fnum), dtype=jnp.int32)
  fid = fid_base[None, :] + y
  gframe = (jnp.arange(B, dtype=jnp.int32)[:, None] * T
            + fid).reshape(-1)
  gidx = jnp.broadcast_to(gframe[:, None],
                          (B * fnum, LANES)).astype(jnp.int32)
  x_m = x.reshape(B * T, *x.shape[2:])
  out = _sc_gather(x_m, gidx)
  return out.reshape(B, fnum, *x.shape[2:])


# speedup vs baseline: 3.4743x; 3.4743x over previous
import functools

import jax
import jax.numpy as jnp
from jax import lax
from jax.experimental import pallas as pl
from jax.experimental.pallas import tpu as pltpu
from jax.experimental.pallas import tpu_sc as plsc

N_B = 8
T_LEN = 128
FNUM = 16
R_PER_B = 3 * 112 * 112
N_ROWS = N_B * R_PER_B
N_TILES = 32
ROWS_PER_TILE_B = R_PER_B // N_TILES
CHUNK_ROWS = 392
CHUNKS_PER_B = ROWS_PER_TILE_B // CHUNK_ROWS
N_CHUNKS = N_B * CHUNKS_PER_B


def _sc_gather(xt2, fid_tbl):
  mesh = plsc.VectorSubcoreMesh(core_axis_name="c", subcore_axis_name="s")

  @functools.partial(
      pl.kernel,
      mesh=mesh,
      out_type=jax.ShapeDtypeStruct((N_ROWS * FNUM,), jnp.float32),
      scratch_types=[
          pltpu.VMEM((N_B, FNUM), jnp.int32),
          pltpu.VMEM((CHUNK_ROWS, T_LEN), jnp.float32),
          pltpu.VMEM((CHUNK_ROWS, T_LEN), jnp.float32),
          pltpu.VMEM((CHUNK_ROWS * FNUM,), jnp.float32),
          pltpu.VMEM((CHUNK_ROWS * FNUM,), jnp.float32),
          pltpu.SemaphoreType.DMA,
          pltpu.SemaphoreType.DMA,
          pltpu.SemaphoreType.DMA,
          pltpu.SemaphoreType.DMA,
      ],
      compiler_params=pltpu.CompilerParams(needs_layout_passes=False),
  )
  def k(xt_hbm, fid_hbm, out_hbm, fid_v, in_a, in_b, out_a, out_b,
        gs_a, gs_b, ss_a, ss_b):
    wid = lax.axis_index("s") * 2 + lax.axis_index("c")
    base = wid * ROWS_PER_TILE_B
    pltpu.sync_copy(fid_hbm, fid_v)
    ins = (in_a, in_b)
    outs = (out_a, out_b)
    gsems = (gs_a, gs_b)
    ssems = (ss_a, ss_b)

    def row0(c):
      b, sub = divmod(c, CHUNKS_PER_B)
      return b * R_PER_B + base + sub * CHUNK_ROWS

    gathers = [None] * N_CHUNKS
    scatters = [None, None]
    gathers[0] = pltpu.async_copy(
        xt_hbm.at[pl.ds(row0(0), CHUNK_ROWS)], ins[0], gsems[0])
    gathers[1] = pltpu.async_copy(
        xt_hbm.at[pl.ds(row0(1), CHUNK_ROWS)], ins[1], gsems[1])
    for c in range(N_CHUNKS):
      slot = c % 2
      b = c // CHUNKS_PER_B
      tvec = fid_v[b, :]
      in_buf, out_buf = ins[slot], outs[slot]
      gathers[c].wait()
      if scatters[slot] is not None:
        scatters[slot].wait()

      @plsc.parallel_loop(0, CHUNK_ROWS, 1, unroll=8)
      def body(w):
        wv = jnp.full((FNUM,), w, jnp.int32)
        vals = plsc.load_gather(in_buf, [wv, tvec])
        out_buf[pl.ds(w * FNUM, FNUM)] = vals

      scatters[slot] = pltpu.async_copy(
          out_buf, out_hbm.at[pl.ds(row0(c) * FNUM, CHUNK_ROWS * FNUM)],
          ssems[slot])
      if c + 2 < N_CHUNKS:
        gathers[c + 2] = pltpu.async_copy(
            xt_hbm.at[pl.ds(row0(c + 2), CHUNK_ROWS)], in_buf, gsems[slot])
    scatters[0].wait()
    scatters[1].wait()

  return k(xt2, fid_tbl)


def kernel(x):
  B, T = x.shape[0], x.shape[1]
  fnum = FNUM
  start, end = 0, T - 1
  fid_base = jnp.linspace(start, end, fnum).astype(jnp.int32)
  step = int((end - start) / fnum)
  if step != 0:
    key = jax.random.key(42)
    y = jax.random.randint(key, (B, fnum), 0, step, dtype=jnp.int32)
    y = y.at[:, fnum - 1].set(0)
  else:
    y = jnp.zeros((B, fnum), dtype=jnp.int32)
  fid = fid_base[None, :] + y
  xt2 = jnp.transpose(x, (0, 2, 3, 4, 1)).reshape(N_ROWS, T_LEN)
  out_flat = _sc_gather(xt2, fid)
  out_t = out_flat.reshape(B, *x.shape[2:], fnum)
  return jnp.transpose(out_t, (0, 4, 1, 2, 3))
